# baseline (device time: 673401 ns/iter reference)
import jax
import jax.numpy as jnp
from jax import lax
from jax.experimental import pallas as pl
from jax.experimental.pallas import tpu as pltpu

N_DEV = 8
_GELU_C = 0.7978845608028654


def _gelu(y):
    return 0.5 * y * (1.0 + jnp.tanh(_GELU_C * (y + 0.044715 * y * y * y)))


def kernel(x, w_mat):
    m_per, k = x.shape
    _, n_per = w_mat.shape

    def body(x_ref, w_ref, out_ref, comm_ref, send_sems, recv_sems):
        my = lax.axis_index("i")
        left = (my - 1) % N_DEV
        right = (my + 1) % N_DEV

        barrier_sem = pltpu.get_barrier_semaphore()
        for nbr in (left, right):
            pl.semaphore_signal(
                barrier_sem, inc=1,
                device_id=(nbr,), device_id_type=pl.DeviceIdType.MESH,
            )
        pl.semaphore_wait(barrier_sem, 2)

        comm_ref[0] = x_ref[...]
        out_ref[pl.ds(my * m_per, m_per), :] = _gelu(
            jnp.dot(x_ref[...], w_ref[...], preferred_element_type=jnp.float32)
        )

        for h in range(N_DEV - 1):
            s = h % 2
            r = (h + 1) % 2
            rdma = pltpu.make_async_remote_copy(
                src_ref=comm_ref.at[s],
                dst_ref=comm_ref.at[r],
                send_sem=send_sems.at[s],
                recv_sem=recv_sems.at[r],
                device_id=(right,),
                device_id_type=pl.DeviceIdType.MESH,
            )
            rdma.start()
            rdma.wait()
            origin = (my - h - 1) % N_DEV
            out_ref[pl.ds(origin * m_per, m_per), :] = _gelu(
                jnp.dot(comm_ref[r], w_ref[...],
                        preferred_element_type=jnp.float32)
            )

    return pl.pallas_call(
        body,
        out_shape=jax.ShapeDtypeStruct((N_DEV * m_per, n_per), jnp.float32),
        in_specs=[
            pl.BlockSpec(memory_space=pltpu.VMEM),
            pl.BlockSpec(memory_space=pltpu.VMEM),
        ],
        out_specs=pl.BlockSpec(memory_space=pltpu.VMEM),
        scratch_shapes=[
            pltpu.VMEM((2, m_per, k), jnp.float32),
            pltpu.SemaphoreType.DMA((2,)),
            pltpu.SemaphoreType.DMA((2,)),
        ],
        compiler_params=pltpu.CompilerParams(collective_id=0),
    )(x, w_mat)


# device time: 345327 ns/iter; 1.9500x vs baseline; 1.9500x over previous
import jax
import jax.numpy as jnp
from jax import lax
from jax.experimental import pallas as pl
from jax.experimental.pallas import tpu as pltpu

N_DEV = 8
N_HOP = N_DEV // 2
_GELU_C = 0.7978845608028654


def _gelu(y):
    return 0.5 * y * (1.0 + jnp.tanh(_GELU_C * (y + 0.044715 * y * y * y)))


def kernel(x, w_mat):
    m_per, k = x.shape
    _, n_per = w_mat.shape
    half = m_per // 2

    def body(x_ref, w_ref, out_ref, cf_ref, cb_ref,
             f_send, f_recv, b_send, b_recv, credit_f, credit_b):
        my = lax.axis_index("i")
        left = (my - 1) % N_DEV
        right = (my + 1) % N_DEV

        def gemm_rows(src, origin, row0, nrows):
            out_ref[pl.ds(origin * m_per + row0, nrows), :] = _gelu(
                jnp.dot(src, w_ref[...], preferred_element_type=jnp.float32)
            )

        barrier_sem = pltpu.get_barrier_semaphore()
        for nbr in (left, right):
            pl.semaphore_signal(
                barrier_sem, inc=1,
                device_id=(nbr,), device_id_type=pl.DeviceIdType.MESH,
            )
        pl.semaphore_wait(barrier_sem, 2)

        cf_ref[0] = x_ref[...]
        cb_ref[0] = x_ref[...]

        for h in range(N_HOP):
            s = h % 2
            r = (h + 1) % 2
            if h >= 1:
                pl.semaphore_wait(credit_f, 1)
                pl.semaphore_wait(credit_b, 1)

            if h < N_HOP - 1:
                f_src, f_dst = cf_ref.at[s], cf_ref.at[r]
                b_src, b_dst = cb_ref.at[s], cb_ref.at[r]
            else:
                f_src = cf_ref.at[s, pl.ds(0, half)]
                f_dst = cf_ref.at[r, pl.ds(0, half)]
                b_src = cb_ref.at[s, pl.ds(half, half)]
                b_dst = cb_ref.at[r, pl.ds(half, half)]

            fwd = pltpu.make_async_remote_copy(
                src_ref=f_src, dst_ref=f_dst,
                send_sem=f_send.at[s], recv_sem=f_recv.at[r],
                device_id=(right,), device_id_type=pl.DeviceIdType.MESH,
            )
            bwd = pltpu.make_async_remote_copy(
                src_ref=b_src, dst_ref=b_dst,
                send_sem=b_send.at[s], recv_sem=b_recv.at[r],
                device_id=(left,), device_id_type=pl.DeviceIdType.MESH,
            )
            fwd.start()
            bwd.start()

            if h == 0:
                gemm_rows(x_ref[...], my, 0, m_per)
            else:
                gemm_rows(cf_ref[s], (my - h) % N_DEV, 0, m_per)
                gemm_rows(cb_ref[s], (my + h) % N_DEV, 0, m_per)

            fwd.wait()
            bwd.wait()

            if h < N_HOP - 1:
                pl.semaphore_signal(
                    credit_f, inc=1,
                    device_id=(left,), device_id_type=pl.DeviceIdType.MESH,
                )
                pl.semaphore_signal(
                    credit_b, inc=1,
                    device_id=(right,), device_id_type=pl.DeviceIdType.MESH,
                )

        last = (my - N_HOP) % N_DEV
        r_last = N_HOP % 2
        gemm_rows(cf_ref[r_last, pl.ds(0, half)], last, 0, half)
        gemm_rows(cb_ref[r_last, pl.ds(half, half)], last, half, half)

    return pl.pallas_call(
        body,
        out_shape=jax.ShapeDtypeStruct((N_DEV * m_per, n_per), jnp.float32),
        in_specs=[
            pl.BlockSpec(memory_space=pltpu.VMEM),
            pl.BlockSpec(memory_space=pltpu.VMEM),
        ],
        out_specs=pl.BlockSpec(memory_space=pltpu.VMEM),
        scratch_shapes=[
            pltpu.VMEM((2, m_per, k), jnp.float32),
            pltpu.VMEM((2, m_per, k), jnp.float32),
            pltpu.SemaphoreType.DMA((2,)),
            pltpu.SemaphoreType.DMA((2,)),
            pltpu.SemaphoreType.DMA((2,)),
            pltpu.SemaphoreType.DMA((2,)),
            pltpu.SemaphoreType.REGULAR,
            pltpu.SemaphoreType.REGULAR,
        ],
        compiler_params=pltpu.CompilerParams(
            collective_id=0,
            vmem_limit_bytes=100 * 1024 * 1024,
        ),
    )(x, w_mat)


# device time: 239068 ns/iter; 2.8168x vs baseline; 1.4445x over previous
import jax
import jax.numpy as jnp
from jax import lax
from jax.experimental import pallas as pl
from jax.experimental.pallas import tpu as pltpu

N_DEV = 8
_GELU_C = 0.7978845608028654

_PART_MASKS = ((1, 3, 4), (3, 4, 1), (4, 1, 3))


def _gelu(y):
    return 0.5 * y * (1.0 + jnp.tanh(_GELU_C * (y + 0.044715 * y * y * y)))


def kernel(x, w_mat):
    m_per, k = x.shape
    _, n_per = w_mat.shape

    base = m_per // 3 // 8 * 8
    part_rows = (m_per - 2 * base, base, base)
    part_offs = (0, part_rows[0], part_rows[0] + base)

    def body(x_ref, w_ref, out_ref, buf_a, buf_b, buf_c,
             send_sems, recv_sems, credits):
        my = lax.axis_index("i")
        bufs = (buf_a, buf_b, buf_c)

        barrier_sem = pltpu.get_barrier_semaphore()
        for mask in (1, 3, 4):
            pl.semaphore_signal(
                barrier_sem, inc=1,
                device_id=(my ^ mask,), device_id_type=pl.DeviceIdType.MESH,
            )
        pl.semaphore_wait(barrier_sem, 3)

        def x_part(p):
            return x_ref.at[pl.ds(part_offs[p], part_rows[p]), :]

        def src_ref(p, b):
            return x_part(p) if b == 0 else bufs[p].at[b - 1]

        def gemm(p, slot_idx, origin):
            out_ref[pl.ds(origin * m_per + part_offs[p], part_rows[p]), :] = (
                _gelu(jnp.dot(bufs[p][slot_idx], w_ref[...],
                              preferred_element_type=jnp.float32))
            )

        def rdma(p, j, b, dst_idx):
            sem = (1 << j) - 1 + b
            return pltpu.make_async_remote_copy(
                src_ref=src_ref(p, b),
                dst_ref=bufs[p].at[dst_idx],
                send_sem=send_sems.at[p, sem],
                recv_sem=recv_sems.at[p, sem],
                device_id=(my ^ _PART_MASKS[p][j],),
                device_id_type=pl.DeviceIdType.MESH,
            )

        masks = _PART_MASKS
        all_rdmas = []

        s0 = [rdma(p, 0, 0, 0) for p in range(3)]
        all_rdmas += s0
        for r in s0:
            r.start()
        out_ref[pl.ds(my * m_per, m_per), :] = _gelu(
            jnp.dot(x_ref[...], w_ref[...], preferred_element_type=jnp.float32)
        )
        for r in s0:
            r.wait_recv()

        s1 = [[rdma(p, 1, b, 1 + b) for b in range(2)] for p in range(3)]
        for p in range(3):
            for b in range(2):
                all_rdmas.append(s1[p][b])
                s1[p][b].start()
        for p in range(3):
            gemm(p, 0, my ^ masks[p][0])
        for p in range(3):
            s1[p][0].wait_recv()
            gemm(p, 1, my ^ masks[p][1])
        for p in range(3):
            s1[p][1].wait_recv()

        s2 = [[rdma(p, 2, b, 3 + b % 2) for b in range(4)] for p in range(3)]
        for p in range(3):
            for b in range(2):
                all_rdmas.append(s2[p][b])
                s2[p][b].start()
        for p in range(3):
            gemm(p, 2, my ^ masks[p][1] ^ masks[p][0])

        for b in range(4):
            for p in range(3):
                m2 = masks[p][2]
                s2[p][b].wait_recv()
                org = (my ^ m2) ^ (0, masks[p][0], masks[p][1],
                                   masks[p][0] ^ masks[p][1])[b]
                gemm(p, 3 + b % 2, org)
                if b < 2:
                    pl.semaphore_signal(
                        credits.at[p], inc=1,
                        device_id=(my ^ m2,),
                        device_id_type=pl.DeviceIdType.MESH,
                    )
            if b < 2:
                for p in range(3):
                    pl.semaphore_wait(credits.at[p], 1)
                    all_rdmas.append(s2[p][b + 2])
                    s2[p][b + 2].start()

        for r in all_rdmas:
            r.wait_send()

    return pl.pallas_call(
        body,
        out_shape=jax.ShapeDtypeStruct((N_DEV * m_per, n_per), jnp.float32),
        in_specs=[
            pl.BlockSpec(memory_space=pltpu.VMEM),
            pl.BlockSpec(memory_space=pltpu.VMEM),
        ],
        out_specs=pl.BlockSpec(memory_space=pltpu.VMEM),
        scratch_shapes=[
            pltpu.VMEM((5, part_rows[0], k), jnp.float32),
            pltpu.VMEM((5, part_rows[1], k), jnp.float32),
            pltpu.VMEM((5, part_rows[2], k), jnp.float32),
            pltpu.SemaphoreType.DMA((3, 7)),
            pltpu.SemaphoreType.DMA((3, 7)),
            pltpu.SemaphoreType.REGULAR((3,)),
        ],
        compiler_params=pltpu.CompilerParams(
            collective_id=0,
            vmem_limit_bytes=100 * 1024 * 1024,
        ),
    )(x, w_mat)


# device time: 231528 ns/iter; 2.9085x vs baseline; 1.0326x over previous
import jax
import jax.numpy as jnp
from jax import lax
from jax.experimental import pallas as pl
from jax.experimental.pallas import tpu as pltpu

N_DEV = 8
_GELU_C = 0.7978845608028654

_PART_MASKS = ((1, 3, 4), (3, 4, 1), (4, 1, 3))


def _gelu(y):
    return 0.5 * y * (1.0 + jnp.tanh(_GELU_C * (y + 0.044715 * y * y * y)))


def kernel(x, w_mat):
    m_per, k = x.shape
    _, n_per = w_mat.shape

    base = m_per // 3 // 8 * 8
    part_rows = (m_per - 2 * base, base, base)
    part_offs = (0, part_rows[0], part_rows[0] + base)

    def body(x_ref, w_ref, out_ref, buf_a, buf_b, buf_c,
             send_sems, recv_sems, credits):
        my = lax.axis_index("i")
        bufs = (buf_a, buf_b, buf_c)

        barrier_sem = pltpu.get_barrier_semaphore()
        for mask in (1, 3, 4):
            pl.semaphore_signal(
                barrier_sem, inc=1,
                device_id=(my ^ mask,), device_id_type=pl.DeviceIdType.MESH,
            )
        pl.semaphore_wait(barrier_sem, 3)

        def x_part(p):
            return x_ref.at[pl.ds(part_offs[p], part_rows[p]), :]

        def src_ref(p, b):
            return x_part(p) if b == 0 else bufs[p].at[b - 1]

        def gemm(p, slot_idx, origin):
            out_ref[pl.ds(origin * m_per + part_offs[p], part_rows[p]), :] = (
                _gelu(jnp.dot(bufs[p][slot_idx], w_ref[...],
                              preferred_element_type=jnp.float32))
            )

        def rdma(p, j, b, dst_idx):
            sem = (1 << j) - 1 + b
            return pltpu.make_async_remote_copy(
                src_ref=src_ref(p, b),
                dst_ref=bufs[p].at[dst_idx],
                send_sem=send_sems.at[p, sem],
                recv_sem=recv_sems.at[p, sem],
                device_id=(my ^ _PART_MASKS[p][j],),
                device_id_type=pl.DeviceIdType.MESH,
            )

        masks = _PART_MASKS
        s0 = [rdma(p, 0, 0, 0) for p in range(3)]
        s1 = [[rdma(p, 1, b, 1 + b) for b in range(2)] for p in range(3)]
        s2 = [[rdma(p, 2, b, 3 + b % 2) for b in range(4)] for p in range(3)]
        all_rdmas = s0 + [r for pp in s1 for r in pp] + \
            [r for pp in s2 for r in pp]

        for p in range(3):
            s0[p].start()
        for p in range(3):
            s1[p][0].start()

        out_ref[pl.ds(my * m_per, m_per), :] = _gelu(
            jnp.dot(x_ref[...], w_ref[...], preferred_element_type=jnp.float32)
        )

        for p in range(3):
            s0[p].wait_recv()
        for p in range(3):
            s1[p][1].start()
        for p in range(3):
            s2[p][0].start()
        for p in range(3):
            s2[p][1].start()
        for p in range(3):
            gemm(p, 0, my ^ masks[p][0])

        for p in range(3):
            s1[p][0].wait_recv()
        for p in range(3):
            gemm(p, 1, my ^ masks[p][1])
        for p in range(3):
            s1[p][1].wait_recv()
        for p in range(3):
            gemm(p, 2, my ^ masks[p][1] ^ masks[p][0])

        for b in range(4):
            for p in range(3):
                m2 = masks[p][2]
                s2[p][b].wait_recv()
                org = (my ^ m2) ^ (0, masks[p][0], masks[p][1],
                                   masks[p][0] ^ masks[p][1])[b]
                gemm(p, 3 + b % 2, org)
                if b < 2:
                    pl.semaphore_signal(
                        credits.at[p], inc=1,
                        device_id=(my ^ m2,),
                        device_id_type=pl.DeviceIdType.MESH,
                    )
            if b < 2:
                for p in range(3):
                    pl.semaphore_wait(credits.at[p], 1)
                    s2[p][b + 2].start()

        for r in all_rdmas:
            r.wait_send()

    return pl.pallas_call(
        body,
        out_shape=jax.ShapeDtypeStruct((N_DEV * m_per, n_per), jnp.float32),
        in_specs=[
            pl.BlockSpec(memory_space=pltpu.VMEM),
            pl.BlockSpec(memory_space=pltpu.VMEM),
        ],
        out_specs=pl.BlockSpec(memory_space=pltpu.VMEM),
        scratch_shapes=[
            pltpu.VMEM((5, part_rows[0], k), jnp.float32),
            pltpu.VMEM((5, part_rows[1], k), jnp.float32),
            pltpu.VMEM((5, part_rows[2], k), jnp.float32),
            pltpu.SemaphoreType.DMA((3, 7)),
            pltpu.SemaphoreType.DMA((3, 7)),
            pltpu.SemaphoreType.REGULAR((3,)),
        ],
        compiler_params=pltpu.CompilerParams(
            collective_id=0,
            vmem_limit_bytes=100 * 1024 * 1024,
        ),
    )(x, w_mat)


# device time: 228318 ns/iter; 2.9494x vs baseline; 1.0141x over previous
import jax
import jax.numpy as jnp
from jax import lax
from jax.experimental import pallas as pl
from jax.experimental.pallas import tpu as pltpu

N_DEV = 8
_GELU_C = 0.7978845608028654

_PART_MASKS = ((1, 3, 4), (3, 4, 1), (4, 1, 3))


def _gelu(y):
    return 0.5 * y * (1.0 + jnp.tanh(_GELU_C * (y + 0.044715 * y * y * y)))


def kernel(x, w_mat):
    m_per, k = x.shape
    _, n_per = w_mat.shape

    base = m_per // 3 // 8 * 8
    part_rows = (m_per - 2 * base, base, base)
    part_offs = (0, part_rows[0], part_rows[0] + base)

    def body(x_ref, w_ref, out_ref, buf_a, buf_b, buf_c,
             send_sems, recv_sems, credits):
        my = lax.axis_index("i")
        bufs = (buf_a, buf_b, buf_c)

        barrier_sem = pltpu.get_barrier_semaphore()
        for mask in (1, 3, 4):
            pl.semaphore_signal(
                barrier_sem, inc=1,
                device_id=(my ^ mask,), device_id_type=pl.DeviceIdType.MESH,
            )
        pl.semaphore_wait(barrier_sem, 3)

        def x_part(p):
            return x_ref.at[pl.ds(part_offs[p], part_rows[p]), :]

        def src_ref(p, b):
            return x_part(p) if b == 0 else bufs[p].at[b - 1]

        def gemm(p, slot_idx, origin):
            out_ref[pl.ds(origin * m_per + part_offs[p], part_rows[p]), :] = (
                _gelu(jnp.dot(bufs[p][slot_idx], w_ref[...],
                              preferred_element_type=jnp.float32))
            )

        def rdma(p, j, b, dst_idx):
            sem = (1 << j) - 1 + b
            return pltpu.make_async_remote_copy(
                src_ref=src_ref(p, b),
                dst_ref=bufs[p].at[dst_idx],
                send_sem=send_sems.at[p, sem],
                recv_sem=recv_sems.at[p, sem],
                device_id=(my ^ _PART_MASKS[p][j],),
                device_id_type=pl.DeviceIdType.MESH,
            )

        masks = _PART_MASKS
        s0 = [rdma(p, 0, 0, 0) for p in range(3)]
        s1 = [[rdma(p, 1, b, 1 + b) for b in range(2)] for p in range(3)]
        s2 = [[rdma(p, 2, b, 3 + b % 2) for b in range(4)] for p in range(3)]
        all_rdmas = s0 + [r for pp in s1 for r in pp] + \
            [r for pp in s2 for r in pp]

        for p in range(3):
            s0[p].start()
        for p in range(3):
            s1[p][0].start()

        out_ref[pl.ds(my * m_per, m_per), :] = _gelu(
            jnp.dot(x_ref[...], w_ref[...], preferred_element_type=jnp.float32)
        )

        for p in range(3):
            s0[p].wait_recv()
        for p in range(3):
            s1[p][1].start()
        for p in range(3):
            s2[p][0].start()
        for p in range(3):
            s2[p][1].start()
        for p in range(3):
            gemm(p, 0, my ^ masks[p][0])

        for p in range(3):
            s1[p][0].wait_recv()
        for p in range(3):
            gemm(p, 1, my ^ masks[p][1])
        for p in range(3):
            s1[p][1].wait_recv()
        for p in range(3):
            gemm(p, 2, my ^ masks[p][1] ^ masks[p][0])

        for b in range(4):
            for p in (1, 2, 0):
                m2 = masks[p][2]
                s2[p][b].wait_recv()
                org = (my ^ m2) ^ (0, masks[p][0], masks[p][1],
                                   masks[p][0] ^ masks[p][1])[b]
                gemm(p, 3 + b % 2, org)
                if b < 2:
                    pl.semaphore_signal(
                        credits.at[p], inc=1,
                        device_id=(my ^ m2,),
                        device_id_type=pl.DeviceIdType.MESH,
                    )
            if b < 2:
                for p in (1, 2, 0):
                    pl.semaphore_wait(credits.at[p], 1)
                    s2[p][b + 2].start()

        for r in all_rdmas:
            r.wait_send()

    return pl.pallas_call(
        body,
        out_shape=jax.ShapeDtypeStruct((N_DEV * m_per, n_per), jnp.float32),
        in_specs=[
            pl.BlockSpec(memory_space=pltpu.VMEM),
            pl.BlockSpec(memory_space=pltpu.VMEM),
        ],
        out_specs=pl.BlockSpec(memory_space=pltpu.VMEM),
        scratch_shapes=[
            pltpu.VMEM((5, part_rows[0], k), jnp.float32),
            pltpu.VMEM((5, part_rows[1], k), jnp.float32),
            pltpu.VMEM((5, part_rows[2], k), jnp.float32),
            pltpu.SemaphoreType.DMA((3, 7)),
            pltpu.SemaphoreType.DMA((3, 7)),
            pltpu.SemaphoreType.REGULAR((3,)),
        ],
        compiler_params=pltpu.CompilerParams(
            collective_id=0,
            vmem_limit_bytes=100 * 1024 * 1024,
        ),
    )(x, w_mat)
